# baseline (speedup 1.0000x reference)
import jax
import jax.numpy as jnp
from jax.experimental import pallas as pl

N_POINTS = 500000
VOXEL_SIZE = 0.1
DT_PREDICTION = 0.1
HIDDEN = 64

_BR = 10000
_GRID = N_POINTS // _BR


def _body(x_ref, q_ref, w1_ref, b1_ref, w2_ref, b2_ref, oc_ref, of_ref):
    q = q_ref[...]
    oc_ref[...] = (x_ref[...] / q) * q
    bf = lambda v: v.astype(jnp.bfloat16).astype(jnp.float32)
    h = jnp.maximum(0.5 * bf(w1_ref[...]) + b1_ref[...], 0.0)
    s = jnp.sum(bf(h) * bf(w2_ref[...])) + b2_ref[0, 0]
    of_ref[...] = jnp.full(of_ref.shape, s, dtype=of_ref.dtype)


def kernel(point_cloud, W1, b1, W2, b2):
    qrow = jnp.array([[VOXEL_SIZE, VOXEL_SIZE, VOXEL_SIZE, DT_PREDICTION]],
                     dtype=point_cloud.dtype)
    w1 = W1.reshape(1, HIDDEN)
    b1r = b1.reshape(1, HIDDEN)
    w2 = W2.reshape(1, HIDDEN)
    b2r = b2.reshape(1, 1)

    full = lambda shape: pl.BlockSpec(shape, lambda i: (0, 0))
    out_coords, out_feats = pl.pallas_call(
        _body,
        grid=(_GRID,),
        in_specs=[
            pl.BlockSpec((_BR, 4), lambda i: (i, 0)),
            full((1, 4)),
            full((1, HIDDEN)),
            full((1, HIDDEN)),
            full((1, HIDDEN)),
            full((1, 1)),
        ],
        out_specs=[
            pl.BlockSpec((_BR, 4), lambda i: (i, 0)),
            pl.BlockSpec((_BR, 1), lambda i: (i, 0)),
        ],
        out_shape=[
            jax.ShapeDtypeStruct((N_POINTS, 4), point_cloud.dtype),
            jax.ShapeDtypeStruct((N_POINTS, 1), point_cloud.dtype),
        ],
    )(point_cloud, qrow, w1, b1r, w2, b2r)
    return out_feats, out_coords
